# baseline (speedup 1.0000x reference)
import jax
import jax.numpy as jnp
from jax.experimental import pallas as pl
from jax.experimental.pallas import tpu as pltpu

_CODEBOOK = 1024
_D = 64
_COMMIT = 0.25


def _vq_body(z_ref, emb_ref, zq_ref, idx_ref, loss_ref):
    b = pl.program_id(0)
    nb = pl.num_programs(0)
    zb = z_ref[0]
    emb = emb_ref[...]
    D, H, W = zb.shape
    P = H * W
    zt = jnp.transpose(zb, (1, 2, 0)).reshape(P, D)
    zsq = jnp.sum(zt * zt, axis=1)
    esq = jnp.sum(emb * emb, axis=1)
    neg2s = jax.lax.dot_general(
        -2.0 * zt, emb, (((1,), (1,)), ((), ())),
        preferred_element_type=jnp.float32)
    dist = (zsq[:, None] + neg2s) + esq[None, :]
    m = jnp.min(dist, axis=1, keepdims=True)
    c_iota = jax.lax.broadcasted_iota(
        jnp.int32, dist.shape, 1).astype(jnp.float32)
    idxf = jnp.min(jnp.where(dist == m, c_iota, jnp.float32(_CODEBOOK)),
                   axis=1)
    idx_ref[0] = idxf.astype(jnp.int32).reshape(H, W)

    onehot = (c_iota == idxf[:, None]).astype(jnp.bfloat16)
    hi = emb.astype(jnp.bfloat16)
    lo = (emb - hi.astype(jnp.float32)).astype(jnp.bfloat16)
    zq = jax.lax.dot_general(
        onehot, hi, (((1,), (0,)), ((), ())),
        preferred_element_type=jnp.float32)
    zq += jax.lax.dot_general(
        onehot, lo, (((1,), (0,)), ((), ())),
        preferred_element_type=jnp.float32)
    zq_ref[0] = jnp.transpose(zq.reshape(H, W, D), (2, 0, 1))

    part = jnp.sum(m)

    @pl.when(b == 0)
    def _init():
        loss_ref[0, 0] = jnp.float32(0.0)

    loss_ref[0, 0] += part

    @pl.when(b == nb - 1)
    def _scale():
        loss_ref[0, 0] *= jnp.float32(_COMMIT / (nb * P * D))


def kernel(z, embedding):
    B, D, H, W = z.shape

    zq, idx, loss_raw = pl.pallas_call(
        _vq_body,
        grid=(B,),
        in_specs=[
            pl.BlockSpec((1, D, H, W), lambda b: (b, 0, 0, 0)),
            pl.BlockSpec((_CODEBOOK, D), lambda b: (0, 0)),
        ],
        out_specs=[
            pl.BlockSpec((1, D, H, W), lambda b: (b, 0, 0, 0)),
            pl.BlockSpec((1, H, W), lambda b: (b, 0, 0)),
            pl.BlockSpec((1, 1), lambda b: (0, 0),
                         memory_space=pltpu.SMEM),
        ],
        out_shape=[
            jax.ShapeDtypeStruct((B, D, H, W), jnp.float32),
            jax.ShapeDtypeStruct((B, H, W), jnp.int32),
            jax.ShapeDtypeStruct((1, 1), jnp.float32),
        ],
    )(z, embedding)

    return (zq, loss_raw[0, 0], idx)


# speedup vs baseline: 1.5485x; 1.5485x over previous
import jax
import jax.numpy as jnp
from jax.experimental import pallas as pl
from jax.experimental.pallas import tpu as pltpu

_CODEBOOK = 1024
_D = 64
_COMMIT = 0.25


def _vq_body(z_ref, emb_ref, zq_ref, idx_ref, loss_ref):
    b = pl.program_id(0)
    nb = pl.num_programs(0)
    zb = z_ref[0]
    emb = emb_ref[...]
    P = zb.shape[1]
    zt = zb.T
    zsq = jnp.sum(zt * zt, axis=1)
    esq = jnp.sum(emb * emb, axis=1)
    neg2s = jax.lax.dot_general(
        -2.0 * zt, emb, (((1,), (1,)), ((), ())),
        preferred_element_type=jnp.float32)
    dist = (zsq[:, None] + neg2s) + esq[None, :]
    m = jnp.min(dist, axis=1, keepdims=True)
    c_iota = jax.lax.broadcasted_iota(
        jnp.int32, dist.shape, 1).astype(jnp.float32)
    idxf = jnp.min(jnp.where(dist == m, c_iota, jnp.float32(_CODEBOOK)),
                   axis=1)
    idx_ref[0, 0, :] = idxf.astype(jnp.int32)

    onehot = (c_iota == idxf[:, None]).astype(jnp.bfloat16)
    hi = emb.astype(jnp.bfloat16)
    lo = (emb - hi.astype(jnp.float32)).astype(jnp.bfloat16)
    zq = jax.lax.dot_general(
        hi, onehot, (((0,), (1,)), ((), ())),
        preferred_element_type=jnp.float32)
    zq += jax.lax.dot_general(
        lo, onehot, (((0,), (1,)), ((), ())),
        preferred_element_type=jnp.float32)
    zq_ref[0] = zq

    part = jnp.sum(m)

    @pl.when(b == 0)
    def _init():
        loss_ref[0, 0] = jnp.float32(0.0)

    loss_ref[0, 0] += part

    @pl.when(b == nb - 1)
    def _scale():
        loss_ref[0, 0] *= jnp.float32(_COMMIT / (nb * P * _D))


def kernel(z, embedding):
    B, D, H, W = z.shape
    P = H * W
    z3 = z.reshape(B, D, P)

    zq3, idx3, loss_raw = pl.pallas_call(
        _vq_body,
        grid=(B,),
        in_specs=[
            pl.BlockSpec((1, D, P), lambda b: (b, 0, 0)),
            pl.BlockSpec((_CODEBOOK, D), lambda b: (0, 0)),
        ],
        out_specs=[
            pl.BlockSpec((1, D, P), lambda b: (b, 0, 0)),
            pl.BlockSpec((1, 1, P), lambda b: (b, 0, 0)),
            pl.BlockSpec((1, 1), lambda b: (0, 0),
                         memory_space=pltpu.SMEM),
        ],
        out_shape=[
            jax.ShapeDtypeStruct((B, D, P), jnp.float32),
            jax.ShapeDtypeStruct((B, 1, P), jnp.int32),
            jax.ShapeDtypeStruct((1, 1), jnp.float32),
        ],
    )(z3, embedding)

    z_q = zq3.reshape(B, D, H, W)
    indices = idx3.reshape(B, H, W)
    return (z_q, loss_raw[0, 0], indices)
